# baseline (speedup 1.0000x reference)
import functools

import jax
import jax.numpy as jnp
from jax import lax
from jax.experimental import pallas as pl
from jax.experimental.pallas import tpu as pltpu
from jax.experimental.pallas import tpu_sc as plsc

N = 10000
E = 320000
D = 128
R = 8
B = 4

NC = 2
NS = 16
NW = NC * NS

EDGES_PER_TILE = E // NW
CHUNK = 128
NCHUNK = -(-EDGES_PER_TILE // CHUNK)
PAD = NCHUNK * CHUNK - EDGES_PER_TILE
NP = N + 16
ROW_BLK = 16
ROWS_PER_TILE = 624


def _tc_prep_body(comps_ref, x_ref, bases_ref, out_ref):
    r = pl.program_id(1)
    w = jnp.zeros((D, D), dtype=jnp.float32)
    for b in range(B):
        w = w + comps_ref[r, b] * bases_ref[b]
    out_ref[...] = jnp.dot(x_ref[...], w, preferred_element_type=jnp.float32)


def _tc_prep(X, bases, comps):
    TN = 400
    grid = (N // TN, R)
    return pl.pallas_call(
        _tc_prep_body,
        grid=grid,
        in_specs=[
            pl.BlockSpec(memory_space=pltpu.SMEM),
            pl.BlockSpec((TN, D), lambda nb, r: (nb, 0)),
            pl.BlockSpec((B, D, D), lambda nb, r: (0, 0, 0)),
        ],
        out_specs=pl.BlockSpec((TN, D), lambda nb, r: (r * (N // TN) + nb, 0)),
        out_shape=jax.ShapeDtypeStruct((R * N, D), jnp.float32),
    )(comps, X, bases)


def _tc_gidx_body(src_ref, typ_ref, out_ref):
    out_ref[...] = typ_ref[...] * N + src_ref[...]


def _tc_gidx(src, typ):
    src2 = src.reshape(E // 128, 128)
    typ2 = typ.reshape(E // 128, 128)
    out = pl.pallas_call(
        _tc_gidx_body,
        out_shape=jax.ShapeDtypeStruct((E // 128, 128), jnp.int32),
    )(src2, typ2)
    return out.reshape(E)


def _pack_chunks(gidx, dst):
    g2 = gidx.reshape(NW, EDGES_PER_TILE)
    d2 = dst.reshape(NW, EDGES_PER_TILE)
    gp = jnp.concatenate(
        [g2, jnp.zeros((NW, PAD), jnp.int32)], axis=1).reshape(NW, NCHUNK, CHUNK)
    dp = jnp.concatenate(
        [d2, jnp.full((NW, PAD), N, jnp.int32)], axis=1).reshape(NW, NCHUNK, CHUNK)
    return jnp.stack([gp, dp], axis=2)


def _sc_edges_body(xw_hbm, idxc_hbm, acc_hbm, deg_hbm,
                   pair_v, rowsA_v, rowsB_v, zrow_v, deg_loc, acc_sh,
                   semA, semB):
    c = lax.axis_index("c")
    s = lax.axis_index("s")
    wid = c * NS + s

    zero16 = jnp.zeros((16,), jnp.float32)
    ones16 = jnp.ones((16,), jnp.float32)

    def init_zrow(i, _):
        zrow_v[i // (D // 16), pl.ds((i % (D // 16)) * 16, 16)] = zero16
        return 0
    lax.fori_loop(0, ROW_BLK * (D // 16), init_zrow, 0)

    def init_deg(i, _):
        deg_loc[pl.ds(i * 16, 16)] = zero16
        return 0
    lax.fori_loop(0, NP // 16, init_deg, 0)

    row0 = s * ROWS_PER_TILE

    def zero_body(i, _):
        pltpu.sync_copy(zrow_v, acc_sh.at[pl.ds(row0 + i * ROW_BLK, ROW_BLK)])
        return 0
    lax.fori_loop(0, ROWS_PER_TILE // ROW_BLK, zero_body, 0)

    @pl.when(s == NS - 1)
    def _():
        base15 = NS * ROWS_PER_TILE
        for j in range((NP - NS * ROWS_PER_TILE) // ROW_BLK):
            pltpu.sync_copy(
                zrow_v, acc_sh.at[pl.ds(base15 + j * ROW_BLK, ROW_BLK)])
    plsc.subcore_barrier()

    def load_idx(g, b):
        pltpu.sync_copy(idxc_hbm.at[wid, g], pair_v.at[b])

    def count_deg(b):
        for i in range(CHUNK // 16):
            idx16 = pair_v[b, 1, pl.ds(i * 16, 16)]
            plsc.addupdate_scatter(deg_loc, [idx16], ones16)

    load_idx(0, 0)
    pltpu.async_copy(xw_hbm.at[pair_v.at[0, 0]], rowsA_v, semA)

    def pair_body(p, _):
        g = p * 2
        load_idx(g + 1, 1)
        pltpu.async_copy(xw_hbm.at[pair_v.at[1, 0]], rowsB_v, semB)
        pltpu.make_async_copy(xw_hbm.at[pair_v.at[0, 0]], rowsA_v, semA).wait()
        pltpu.sync_copy(rowsA_v, acc_sh.at[pair_v.at[0, 1]], add=True)
        count_deg(0)

        @pl.when(g + 2 < NCHUNK)
        def _():
            load_idx(g + 2, 0)
            pltpu.async_copy(xw_hbm.at[pair_v.at[0, 0]], rowsA_v, semA)
        pltpu.make_async_copy(xw_hbm.at[pair_v.at[1, 0]], rowsB_v, semB).wait()
        pltpu.sync_copy(rowsB_v, acc_sh.at[pair_v.at[1, 1]], add=True)
        count_deg(1)
        return 0

    lax.fori_loop(0, NCHUNK // 2, pair_body, 0)

    if NCHUNK % 2 == 1:
        pltpu.make_async_copy(xw_hbm.at[pair_v.at[0, 0]], rowsA_v, semA).wait()
        pltpu.sync_copy(rowsA_v, acc_sh.at[pair_v.at[0, 1]], add=True)
        count_deg(0)

    pltpu.sync_copy(deg_loc, deg_hbm.at[c, s])
    plsc.subcore_barrier()

    def out_body(i, _):
        sl = pl.ds(row0 + i * ROW_BLK, ROW_BLK)
        pltpu.sync_copy(acc_sh.at[sl], acc_hbm.at[c, sl])
        return 0
    lax.fori_loop(0, ROWS_PER_TILE // ROW_BLK, out_body, 0)

    @pl.when(s == NS - 1)
    def _():
        sl = pl.ds(NS * ROWS_PER_TILE, ROW_BLK)
        pltpu.sync_copy(acc_sh.at[sl], acc_hbm.at[c, sl])


@functools.partial(
    pl.kernel,
    out_type=(
        jax.ShapeDtypeStruct((NC, N, D), jnp.float32),
        jax.ShapeDtypeStruct((NC, NS, NP), jnp.float32),
    ),
    mesh=plsc.VectorSubcoreMesh(core_axis_name="c", subcore_axis_name="s",
                                num_cores=NC, num_subcores=NS),
    compiler_params=pltpu.CompilerParams(needs_layout_passes=False),
    scratch_types=[
        pltpu.VMEM((2, 2, CHUNK), jnp.int32),
        pltpu.VMEM((CHUNK, D), jnp.float32),
        pltpu.VMEM((CHUNK, D), jnp.float32),
        pltpu.VMEM((ROW_BLK, D), jnp.float32),
        pltpu.VMEM((NP,), jnp.float32),
        pltpu.VMEM_SHARED((NP, D), jnp.float32),
        pltpu.SemaphoreType.DMA,
        pltpu.SemaphoreType.DMA,
    ],
)
def _sc_edges(xw_hbm, idxc_hbm, acc_hbm, deg_hbm,
              pair_v, rowsA_v, rowsB_v, zrow_v, deg_loc, acc_sh,
              semA, semB):
    _sc_edges_body(xw_hbm, idxc_hbm, acc_hbm, deg_hbm,
                   pair_v, rowsA_v, rowsB_v, zrow_v, deg_loc, acc_sh,
                   semA, semB)


def _tc_degsum_body(deg_ref, out_ref):
    out_ref[...] = jnp.sum(deg_ref[...], axis=0)[:, None]


def _tc_degsum(deg):
    return pl.pallas_call(
        _tc_degsum_body,
        out_shape=jax.ShapeDtypeStruct((NP, 1), jnp.float32),
    )(deg.reshape(NC * NS, NP))


def _tc_finish_body(acc_ref, deg_ref, bias_ref, out_ref):
    acc = acc_ref[0] + acc_ref[1]
    norm = 1.0 / jnp.clip(deg_ref[...], 1.0, None)
    out_ref[...] = jnp.maximum(acc * norm + bias_ref[...], 0.0)


def _tc_finish(acc, deg, bias):
    TN = 400
    grid = (N // TN,)
    return pl.pallas_call(
        _tc_finish_body,
        grid=grid,
        in_specs=[
            pl.BlockSpec((NC, TN, D), lambda nb: (0, nb, 0)),
            pl.BlockSpec((TN, 1), lambda nb: (nb, 0)),
            pl.BlockSpec((1, D), lambda nb: (0, 0)),
        ],
        out_specs=pl.BlockSpec((TN, D), lambda nb: (nb, 0)),
        out_shape=jax.ShapeDtypeStruct((N, D), jnp.float32),
    )(acc, deg, bias.reshape(1, D))


def kernel(X, edge_index, edge_type, bases, comps, bias):
    src = edge_index[0]
    dst = edge_index[1]
    xw = _tc_prep(X, bases, comps)
    gidx = _tc_gidx(src, edge_type)
    idxc = _pack_chunks(gidx, dst)
    acc, deg = _sc_edges(xw, idxc)
    degsum = _tc_degsum(deg)
    return _tc_finish(acc, degsum, bias)


# speedup vs baseline: 1.2932x; 1.2932x over previous
import functools

import jax
import jax.numpy as jnp
from jax import lax
from jax.experimental import pallas as pl
from jax.experimental.pallas import tpu as pltpu
from jax.experimental.pallas import tpu_sc as plsc

N = 10000
E = 320000
D = 128
R = 8
B = 4

NC = 2
NS = 16
NW = NC * NS

EDGES_PER_TILE = E // NW
CHUNK = 80
NCHUNK = -(-EDGES_PER_TILE // CHUNK)
PAD = NCHUNK * CHUNK - EDGES_PER_TILE
NP = N + 16
ROW_BLK = 16
ROWS_PER_TILE = 624


def _tc_prep_body(comps_ref, x_ref, bases_ref, out_ref):
    r = pl.program_id(1)
    w = jnp.zeros((D, D), dtype=jnp.float32)
    for b in range(B):
        w = w + comps_ref[r, b] * bases_ref[b]
    out_ref[...] = jnp.dot(x_ref[...], w, preferred_element_type=jnp.float32)


def _tc_prep(X, bases, comps):
    TN = 400
    grid = (N // TN, R)
    return pl.pallas_call(
        _tc_prep_body,
        grid=grid,
        in_specs=[
            pl.BlockSpec(memory_space=pltpu.SMEM),
            pl.BlockSpec((TN, D), lambda nb, r: (nb, 0)),
            pl.BlockSpec((B, D, D), lambda nb, r: (0, 0, 0)),
        ],
        out_specs=pl.BlockSpec((TN, D), lambda nb, r: (r * (N // TN) + nb, 0)),
        out_shape=jax.ShapeDtypeStruct((R * N, D), jnp.float32),
    )(comps, X, bases)


def _tc_gidx_body(src_ref, typ_ref, out_ref):
    out_ref[...] = typ_ref[...] * N + src_ref[...]


def _tc_gidx(src, typ):
    src2 = src.reshape(E // 128, 128)
    typ2 = typ.reshape(E // 128, 128)
    out = pl.pallas_call(
        _tc_gidx_body,
        out_shape=jax.ShapeDtypeStruct((E // 128, 128), jnp.int32),
    )(src2, typ2)
    return out.reshape(E)


def _pack_chunks(gidx, dst):
    g2 = gidx.reshape(NW, EDGES_PER_TILE)
    d2 = dst.reshape(NW, EDGES_PER_TILE)
    if PAD:
        g2 = jnp.concatenate([g2, jnp.zeros((NW, PAD), jnp.int32)], axis=1)
        d2 = jnp.concatenate([d2, jnp.full((NW, PAD), N, jnp.int32)], axis=1)
    gp = g2.reshape(NW, NCHUNK, CHUNK)
    dp = d2.reshape(NW, NCHUNK, CHUNK)
    return jnp.stack([gp, dp], axis=2)


def _sc_edges_body(xw_hbm, idxc_hbm, acc_hbm, deg_hbm,
                   pair_v, rowsA_v, rowsB_v, zrow_v, deg_loc, acc_sh,
                   semA, semB):
    c = lax.axis_index("c")
    s = lax.axis_index("s")
    wid = c * NS + s

    zero16 = jnp.zeros((16,), jnp.float32)
    ones16 = jnp.ones((16,), jnp.float32)

    def init_zrow(i, _):
        zrow_v[i // (D // 16), pl.ds((i % (D // 16)) * 16, 16)] = zero16
        return 0
    lax.fori_loop(0, ROW_BLK * (D // 16), init_zrow, 0)

    def init_deg(i, _):
        deg_loc[pl.ds(i * 16, 16)] = zero16
        return 0
    lax.fori_loop(0, NP // 16, init_deg, 0)

    row0 = s * ROWS_PER_TILE

    def zero_body(i, _):
        pltpu.sync_copy(zrow_v, acc_sh.at[pl.ds(row0 + i * ROW_BLK, ROW_BLK)])
        return 0
    lax.fori_loop(0, ROWS_PER_TILE // ROW_BLK, zero_body, 0)

    @pl.when(s == NS - 1)
    def _():
        base15 = NS * ROWS_PER_TILE
        for j in range((NP - NS * ROWS_PER_TILE) // ROW_BLK):
            pltpu.sync_copy(
                zrow_v, acc_sh.at[pl.ds(base15 + j * ROW_BLK, ROW_BLK)])
    plsc.subcore_barrier()

    def load_idx(g, b):
        pltpu.sync_copy(idxc_hbm.at[wid, g], pair_v.at[b])

    def count_deg(b):
        for i in range(CHUNK // 16):
            idx16 = pair_v[b, 1, pl.ds(i * 16, 16)]
            plsc.addupdate_scatter(deg_loc, [idx16], ones16)

    load_idx(0, 0)
    pltpu.async_copy(xw_hbm.at[pair_v.at[0, 0]], rowsA_v, semA)

    def pair_body(p, _):
        g = p * 2
        load_idx(g + 1, 1)
        pltpu.async_copy(xw_hbm.at[pair_v.at[1, 0]], rowsB_v, semB)
        pltpu.make_async_copy(xw_hbm.at[pair_v.at[0, 0]], rowsA_v, semA).wait()
        pltpu.sync_copy(rowsA_v, acc_sh.at[pair_v.at[0, 1]], add=True)
        count_deg(0)

        @pl.when(g + 2 < NCHUNK)
        def _():
            load_idx(g + 2, 0)
            pltpu.async_copy(xw_hbm.at[pair_v.at[0, 0]], rowsA_v, semA)
        pltpu.make_async_copy(xw_hbm.at[pair_v.at[1, 0]], rowsB_v, semB).wait()
        pltpu.sync_copy(rowsB_v, acc_sh.at[pair_v.at[1, 1]], add=True)
        count_deg(1)
        return 0

    lax.fori_loop(0, NCHUNK // 2, pair_body, 0)

    if NCHUNK % 2 == 1:
        pltpu.make_async_copy(xw_hbm.at[pair_v.at[0, 0]], rowsA_v, semA).wait()
        pltpu.sync_copy(rowsA_v, acc_sh.at[pair_v.at[0, 1]], add=True)
        count_deg(0)

    pltpu.sync_copy(deg_loc, deg_hbm.at[c, s])
    plsc.subcore_barrier()

    def out_body(i, _):
        sl = pl.ds(row0 + i * ROW_BLK, ROW_BLK)
        pltpu.sync_copy(acc_sh.at[sl], acc_hbm.at[c, sl])
        return 0
    lax.fori_loop(0, ROWS_PER_TILE // ROW_BLK, out_body, 0)

    @pl.when(s == NS - 1)
    def _():
        sl = pl.ds(NS * ROWS_PER_TILE, ROW_BLK)
        pltpu.sync_copy(acc_sh.at[sl], acc_hbm.at[c, sl])


@functools.partial(
    pl.kernel,
    out_type=(
        jax.ShapeDtypeStruct((NC, N, D), jnp.float32),
        jax.ShapeDtypeStruct((NC, NS, NP), jnp.float32),
    ),
    mesh=plsc.VectorSubcoreMesh(core_axis_name="c", subcore_axis_name="s",
                                num_cores=NC, num_subcores=NS),
    compiler_params=pltpu.CompilerParams(needs_layout_passes=False),
    scratch_types=[
        pltpu.VMEM((2, 2, CHUNK), jnp.int32),
        pltpu.VMEM((CHUNK, D), jnp.float32),
        pltpu.VMEM((CHUNK, D), jnp.float32),
        pltpu.VMEM((ROW_BLK, D), jnp.float32),
        pltpu.VMEM((NP,), jnp.float32),
        pltpu.VMEM_SHARED((NP, D), jnp.float32),
        pltpu.SemaphoreType.DMA,
        pltpu.SemaphoreType.DMA,
    ],
)
def _sc_edges(xw_hbm, idxc_hbm, acc_hbm, deg_hbm,
              pair_v, rowsA_v, rowsB_v, zrow_v, deg_loc, acc_sh,
              semA, semB):
    _sc_edges_body(xw_hbm, idxc_hbm, acc_hbm, deg_hbm,
                   pair_v, rowsA_v, rowsB_v, zrow_v, deg_loc, acc_sh,
                   semA, semB)


def _tc_degsum_body(deg_ref, out_ref):
    out_ref[...] = jnp.sum(deg_ref[...], axis=0)[:, None]


def _tc_degsum(deg):
    return pl.pallas_call(
        _tc_degsum_body,
        out_shape=jax.ShapeDtypeStruct((NP, 1), jnp.float32),
    )(deg.reshape(NC * NS, NP))


def _tc_finish_body(acc_ref, deg_ref, bias_ref, out_ref):
    acc = acc_ref[0] + acc_ref[1]
    norm = 1.0 / jnp.clip(deg_ref[...], 1.0, None)
    out_ref[...] = jnp.maximum(acc * norm + bias_ref[...], 0.0)


def _tc_finish(acc, deg, bias):
    TN = 400
    grid = (N // TN,)
    return pl.pallas_call(
        _tc_finish_body,
        grid=grid,
        in_specs=[
            pl.BlockSpec((NC, TN, D), lambda nb: (0, nb, 0)),
            pl.BlockSpec((TN, 1), lambda nb: (nb, 0)),
            pl.BlockSpec((1, D), lambda nb: (0, 0)),
        ],
        out_specs=pl.BlockSpec((TN, D), lambda nb: (nb, 0)),
        out_shape=jax.ShapeDtypeStruct((N, D), jnp.float32),
    )(acc, deg, bias.reshape(1, D))


def kernel(X, edge_index, edge_type, bases, comps, bias):
    src = edge_index[0]
    dst = edge_index[1]
    xw = _tc_prep(X, bases, comps)
    gidx = _tc_gidx(src, edge_type)
    idxc = _pack_chunks(gidx, dst)
    acc, deg = _sc_edges(xw, idxc)
    degsum = _tc_degsum(deg)
    return _tc_finish(acc, degsum, bias)
